# baseline (speedup 1.0000x reference)
import functools

import jax
import jax.numpy as jnp
from jax import lax
from jax.experimental import pallas as pl
from jax.experimental.pallas import tpu as pltpu
from jax.experimental.pallas import tpu_sc as plsc

_NC = 2
_NS = 16
_NW = _NC * _NS

_NBUF = 2

_MESH = plsc.VectorSubcoreMesh(core_axis_name="c", subcore_axis_name="s")


def _gather_rows(B: int, S: int, D: int, wlin, flat_idx):
    total = B * S
    b_per_w = total // _NW
    nb_w = B // _NW

    @functools.partial(
        pl.kernel,
        out_type=jax.ShapeDtypeStruct((B, S, D), jnp.float32),
        mesh=_MESH,
        scratch_types=[
            pltpu.VMEM((b_per_w,), jnp.int32),
            [pltpu.VMEM((S, 128), jnp.float32) for _ in range(_NBUF)],
            [pltpu.VMEM((1, S, D), jnp.float32) for _ in range(_NBUF)],
            [pltpu.SemaphoreType.DMA for _ in range(_NBUF)],
            [pltpu.SemaphoreType.DMA for _ in range(_NBUF)],
        ],
    )
    def k2(wlin_hbm, idx_hbm, out_hbm, idx_v, rows, cbuf, sg, sw):
        wid = lax.axis_index("s") * _NC + lax.axis_index("c")
        base = wid * b_per_w
        bbase = wid * nb_w
        pltpu.sync_copy(idx_hbm.at[pl.ds(base, b_per_w)], idx_v)

        def fire_gathers(g, b):
            off = g * S
            cps = []
            done = 0
            while done < S:
                n = min(128, S - done)
                cps.append(pltpu.async_copy(
                    wlin_hbm.at[idx_v.at[pl.ds(off + done, n)]],
                    rows[b].at[pl.ds(done, n)],
                    sg[b],
                ))
                done += n
            return cps

        def repack(b):
            def row(r, carry):
                for kk in range(D // 16):
                    sl = pl.ds(kk * 16, 16)
                    cbuf[b][0, r, sl] = rows[b][r, sl]
                return carry
            lax.fori_loop(0, S, row, 0)

        def wait_write(b):
            pltpu.make_async_copy(
                cbuf[b], out_hbm.at[pl.ds(bbase, 1)], sw[b]
            ).wait()

        def body(t, carry):
            @pl.when(t > 0)
            def _():
                for b in range(_NBUF):
                    wait_write(b)

            copies = []
            for b in range(_NBUF):
                copies.append(fire_gathers(t * _NBUF + b, b))
            for b in range(_NBUF):
                g = t * _NBUF + b
                for c in copies[b]:
                    c.wait()
                repack(b)
                pltpu.async_copy(
                    cbuf[b], out_hbm.at[pl.ds(bbase + g, 1)], sw[b]
                )
            return carry

        lax.fori_loop(0, nb_w // _NBUF, body, 0)
        for b in range(_NBUF):
            wait_write(b)

    return k2(wlin, flat_idx)


def kernel(x, weight):
    B, S = x.shape
    V, D = weight.shape
    wlin = jnp.pad(weight, ((0, 0), (0, 128 - D)))
    flat_idx = x.reshape(B * S).astype(jnp.int32)
    return _gather_rows(B, S, D, wlin, flat_idx)
